# baseline (speedup 1.0000x reference)
import functools

import jax
import jax.numpy as jnp
from jax import lax
from jax.experimental import pallas as pl
from jax.experimental.pallas import tpu as pltpu
from jax.experimental.pallas import tpu_sc as plsc

N = 10000
D = 128
NP = 10240
NC = 2
NS = 16
CHUNK = 128
RPT = NP // NS

@functools.cache
def _get_mesh():
    return plsc.VectorSubcoreMesh(core_axis_name="c", subcore_axis_name="s",
                                  num_cores=NC, num_subcores=NS)


def _chunk_split(e, f0):
    ct = -(-e // (2 * NS * CHUNK)) * (2 * NS)
    per_tile = ct // NS
    n0 = max(2, min(per_tile - 2, 2 * round(f0 * per_tile / 2.0)))
    return n0, per_tile - n0


def _make_deg_kernel(NCHT):
    @functools.partial(
        pl.kernel,
        out_type=jax.ShapeDtypeStruct((NC, NP, D), jnp.float32),
        mesh=_get_mesh(),
        scratch_types=[
            pltpu.VMEM((CHUNK,), jnp.int32),
            pltpu.VMEM((CHUNK,), jnp.int32),
            pltpu.VMEM((CHUNK, D), jnp.float32),
            pltpu.VMEM_SHARED((NP, D), jnp.float32),
            pltpu.SemaphoreType.DMA,
            pltpu.SemaphoreType.DMA,
            pltpu.SemaphoreType.DMA,
            pltpu.SemaphoreType.DMA,
        ],
    )
    def deg_kernel(dst_hbm, zdeg_hbm, ones_hbm, out_hbm,
                   dstA, dstB, ones_v, deg_sp, iA, iB, sA, sB):
        c = lax.axis_index("c")
        s = lax.axis_index("s")
        w = c * NS + s
        pltpu.sync_copy(zdeg_hbm, deg_sp.at[pl.ds(s * RPT, RPT)])
        pltpu.sync_copy(ones_hbm, ones_v)
        tile_base = w * NCHT * CHUNK

        def idx_start(buf, sem, k):
            pltpu.async_copy(dst_hbm.at[pl.ds(tile_base + k * CHUNK, CHUNK)],
                             buf, sem)

        def idx_wait(buf, sem):
            pltpu.make_async_copy(dst_hbm.at[pl.ds(0, CHUNK)], buf, sem).wait()

        def s_start(buf, sem):
            pltpu.async_copy(ones_v, deg_sp.at[buf], sem, add=True)

        def s_wait(buf, sem):
            pltpu.make_async_copy(ones_v, deg_sp.at[buf], sem).wait()

        plsc.subcore_barrier()
        idx_start(dstA, iA, 0)
        idx_start(dstB, iB, 1)
        NH2 = NCHT // 2

        def body(gg, carry):
            k0 = 2 * gg
            idx_wait(dstA, iA)
            s_start(dstA, sA)
            idx_wait(dstB, iB)
            s_start(dstB, sB)
            s_wait(dstA, sA)

            @pl.when(k0 + 2 < NCHT)
            def _():
                idx_start(dstA, iA, k0 + 2)

            s_wait(dstB, sB)

            @pl.when(k0 + 3 < NCHT)
            def _():
                idx_start(dstB, iB, k0 + 3)

            return carry

        lax.fori_loop(0, NH2, body, 0)
        if NCHT % 2 == 1:
            idx_wait(dstA, iA)
            s_start(dstA, sA)
            s_wait(dstA, sA)
        plsc.subcore_barrier()
        pltpu.sync_copy(deg_sp.at[pl.ds(s * RPT, RPT)],
                        out_hbm.at[c, pl.ds(s * RPT, RPT)])

    return deg_kernel


def _make_prop_kernel(n0, n1):
    assert n0 % 2 == 0 and n1 % 2 == 0

    @functools.partial(
        pl.kernel,
        out_type=jax.ShapeDtypeStruct((NC, NP, D), jnp.float32),
        mesh=_get_mesh(),
        scratch_types=[
            pltpu.VMEM((CHUNK,), jnp.int32),
            pltpu.VMEM((CHUNK,), jnp.int32),
            pltpu.VMEM((CHUNK,), jnp.int32),
            pltpu.VMEM((CHUNK,), jnp.int32),
            pltpu.VMEM((CHUNK, D), jnp.float32),
            pltpu.VMEM((CHUNK, D), jnp.float32),
            pltpu.VMEM_SHARED((NP, D), jnp.float32),
            pltpu.SemaphoreType.DMA,
            pltpu.SemaphoreType.DMA,
            pltpu.SemaphoreType.DMA,
            pltpu.SemaphoreType.DMA,
            pltpu.SemaphoreType.DMA,
            pltpu.SemaphoreType.DMA,
        ],
    )
    def prop_kernel(src_hbm, dst_hbm, g_hbm, zrow_hbm, out_hbm,
                    srcA, srcB, dstA, dstB, rowsA, rowsB, acc_sp,
                    iA, iB, gA, gB, sA, sB):
        c = lax.axis_index("c")
        s = lax.axis_index("s")
        nct = jnp.where(c == 0, n0, n1)
        base_chunk = jnp.where(c == 0, s * n0, NS * n0 + s * n1)
        tile_base = base_chunk * CHUNK
        pltpu.sync_copy(zrow_hbm, acc_sp.at[pl.ds(s * RPT, RPT)])

        def idx_start(sbuf, dbuf, sem, k):
            base = tile_base + k * CHUNK
            pltpu.async_copy(src_hbm.at[pl.ds(base, CHUNK)], sbuf, sem)
            pltpu.async_copy(dst_hbm.at[pl.ds(base, CHUNK)], dbuf, sem)

        def idx_wait(sbuf, dbuf, sem):
            pltpu.make_async_copy(src_hbm.at[pl.ds(0, CHUNK)], sbuf, sem).wait()
            pltpu.make_async_copy(dst_hbm.at[pl.ds(0, CHUNK)], dbuf, sem).wait()

        def g_start(sbuf, rbuf, sem):
            pltpu.async_copy(g_hbm.at[sbuf], rbuf, sem)

        def g_wait(sbuf, rbuf, sem):
            pltpu.make_async_copy(g_hbm.at[sbuf], rbuf, sem).wait()

        def s_start(dbuf, rbuf, sem):
            pltpu.async_copy(rbuf, acc_sp.at[dbuf], sem, add=True)

        def s_wait(dbuf, rbuf, sem):
            pltpu.make_async_copy(rbuf, acc_sp.at[dbuf], sem).wait()

        plsc.subcore_barrier()
        idx_start(srcA, dstA, iA, 0)
        idx_start(srcB, dstB, iB, 1)

        def body(gg, carry):
            k0 = 2 * gg
            idx_wait(srcA, dstA, iA)
            g_start(srcA, rowsA, gA)
            idx_wait(srcB, dstB, iB)
            g_start(srcB, rowsB, gB)
            g_wait(srcA, rowsA, gA)
            s_start(dstA, rowsA, sA)
            g_wait(srcB, rowsB, gB)
            s_start(dstB, rowsB, sB)
            s_wait(dstA, rowsA, sA)

            @pl.when(k0 + 2 < nct)
            def _():
                idx_start(srcA, dstA, iA, k0 + 2)

            s_wait(dstB, rowsB, sB)

            @pl.when(k0 + 3 < nct)
            def _():
                idx_start(srcB, dstB, iB, k0 + 3)

            return carry

        lax.fori_loop(0, nct // 2, body, 0)
        plsc.subcore_barrier()
        pltpu.sync_copy(acc_sp.at[pl.ds(s * RPT, RPT)],
                        out_hbm.at[c, pl.ds(s * RPT, RPT)])

    return prop_kernel


_BR = 2048
_GRID = NP // _BR


def _dinv_from_partials(dp):
    s = jnp.sum(dp, axis=0)
    s = jnp.sum(s, axis=1, keepdims=True)
    return lax.rsqrt(s * (1.0 / 128.0) + 1.0)


def _tc_a_body(dp_ref, x_ref, w_ref, g_ref):
    dinv = _dinv_from_partials(dp_ref[...])
    g_ref[...] = jnp.dot(x_ref[...], w_ref[...],
                         preferred_element_type=jnp.float32) * dinv


def _tc_b_body(dp_ref, p_ref, g_ref, b_ref, w_ref, out_ref):
    dinv = _dinv_from_partials(dp_ref[...])
    p = p_ref[...]
    t = (p[0] + p[1] + g_ref[...]) * dinv + b_ref[...]
    z = jnp.maximum(t, 0.0)
    out_ref[...] = jnp.dot(z, w_ref[...],
                           preferred_element_type=jnp.float32) * dinv


def _tc_c_body(dp_ref, p_ref, g_ref, b_ref, out_ref):
    dinv = _dinv_from_partials(dp_ref[...])
    p = p_ref[...]
    out_ref[...] = (p[0] + p[1] + g_ref[...]) * dinv + b_ref[...]


_dp_spec = pl.BlockSpec((NC, _BR, D), lambda i: (0, i, 0))
_row_spec = pl.BlockSpec((_BR, D), lambda i: (i, 0))
_p_spec = pl.BlockSpec((NC, _BR, D), lambda i: (0, i, 0))
_w_spec = pl.BlockSpec((D, D), lambda i: (0, 0))
_b_spec = pl.BlockSpec((1, D), lambda i: (0, 0))
_out_shape = jax.ShapeDtypeStruct((NP, D), jnp.float32)

_tc_a = pl.pallas_call(
    _tc_a_body, grid=(_GRID,),
    in_specs=[_dp_spec, _row_spec, _w_spec],
    out_specs=_row_spec, out_shape=_out_shape)

_tc_b = pl.pallas_call(
    _tc_b_body, grid=(_GRID,),
    in_specs=[_dp_spec, _p_spec, _row_spec, _b_spec, _w_spec],
    out_specs=_row_spec, out_shape=_out_shape)

_tc_c = pl.pallas_call(
    _tc_c_body, grid=(_GRID,),
    in_specs=[_dp_spec, _p_spec, _row_spec, _b_spec],
    out_specs=_row_spec, out_shape=_out_shape)


_F0 = 0.84


def kernel(x, edge_index, W1, b1, W2, b2):
    e = edge_index.shape[1]
    n0, n1 = _chunk_split(e, _F0)
    ep = NS * (n0 + n1) * CHUNK
    ncht_deg = ep // (NC * NS * CHUNK)

    src = edge_index[0].astype(jnp.int32)
    dst = edge_index[1].astype(jnp.int32)
    pad = jnp.full((ep - e,), N, dtype=jnp.int32)
    srcp = jnp.concatenate([src, pad])
    dstp = jnp.concatenate([dst, pad])
    xp = jnp.pad(x, ((0, NP - x.shape[0]), (0, 0)))

    ones_c = jnp.ones((CHUNK, D), jnp.float32)
    zrow = jnp.zeros((RPT, D), jnp.float32)

    dp = _make_deg_kernel(ncht_deg)(dstp, zrow, ones_c)
    g1 = _tc_a(dp, xp, W1)
    prop = _make_prop_kernel(n0, n1)
    p1 = prop(srcp, dstp, g1, zrow)
    g2 = _tc_b(dp, p1, g1, b1.reshape(1, D), W2)
    p2 = prop(srcp, dstp, g2, zrow)
    out = _tc_c(dp, p2, g2, b2.reshape(1, D))
    return out[:N]


# speedup vs baseline: 1.0477x; 1.0477x over previous
import functools

import jax
import jax.numpy as jnp
from jax import lax
from jax.experimental import pallas as pl
from jax.experimental.pallas import tpu as pltpu
from jax.experimental.pallas import tpu_sc as plsc

N = 10000
D = 128
NP = 10240
NC = 2
NS = 16
CHUNK = 128
RPT = NP // NS

@functools.cache
def _get_mesh():
    return plsc.VectorSubcoreMesh(core_axis_name="c", subcore_axis_name="s",
                                  num_cores=NC, num_subcores=NS)


def _chunk_split(e, f0):
    ct = -(-e // (2 * NS * CHUNK)) * (2 * NS)
    per_tile = ct // NS
    n0 = max(2, min(per_tile - 2, 2 * round(f0 * per_tile / 2.0)))
    return n0, per_tile - n0


def _make_deg_kernel(NCHT):
    @functools.partial(
        pl.kernel,
        out_type=jax.ShapeDtypeStruct((NC, NP, D), jnp.float32),
        mesh=_get_mesh(),
        scratch_types=[
            pltpu.VMEM((CHUNK,), jnp.int32),
            pltpu.VMEM((CHUNK,), jnp.int32),
            pltpu.VMEM((CHUNK, D), jnp.float32),
            pltpu.VMEM_SHARED((NP, D), jnp.float32),
            pltpu.SemaphoreType.DMA,
            pltpu.SemaphoreType.DMA,
            pltpu.SemaphoreType.DMA,
            pltpu.SemaphoreType.DMA,
        ],
    )
    def deg_kernel(dst_hbm, zdeg_hbm, ones_hbm, out_hbm,
                   dstA, dstB, ones_v, deg_sp, iA, iB, sA, sB):
        c = lax.axis_index("c")
        s = lax.axis_index("s")
        w = c * NS + s
        pltpu.sync_copy(zdeg_hbm, deg_sp.at[pl.ds(s * RPT, RPT)])
        pltpu.sync_copy(ones_hbm, ones_v)
        tile_base = w * NCHT * CHUNK

        def idx_start(buf, sem, k):
            pltpu.async_copy(dst_hbm.at[pl.ds(tile_base + k * CHUNK, CHUNK)],
                             buf, sem)

        def idx_wait(buf, sem):
            pltpu.make_async_copy(dst_hbm.at[pl.ds(0, CHUNK)], buf, sem).wait()

        def s_start(buf, sem):
            pltpu.async_copy(ones_v, deg_sp.at[buf], sem, add=True)

        def s_wait(buf, sem):
            pltpu.make_async_copy(ones_v, deg_sp.at[buf], sem).wait()

        plsc.subcore_barrier()
        idx_start(dstA, iA, 0)
        idx_start(dstB, iB, 1)
        NH2 = NCHT // 2

        def body(gg, carry):
            k0 = 2 * gg
            idx_wait(dstA, iA)
            s_start(dstA, sA)
            idx_wait(dstB, iB)
            s_start(dstB, sB)
            s_wait(dstA, sA)

            @pl.when(k0 + 2 < NCHT)
            def _():
                idx_start(dstA, iA, k0 + 2)

            s_wait(dstB, sB)

            @pl.when(k0 + 3 < NCHT)
            def _():
                idx_start(dstB, iB, k0 + 3)

            return carry

        lax.fori_loop(0, NH2, body, 0)
        if NCHT % 2 == 1:
            idx_wait(dstA, iA)
            s_start(dstA, sA)
            s_wait(dstA, sA)
        plsc.subcore_barrier()
        pltpu.sync_copy(deg_sp.at[pl.ds(s * RPT, RPT)],
                        out_hbm.at[c, pl.ds(s * RPT, RPT)])

    return deg_kernel


def _make_prop_kernel(n0, n1):
    assert n0 % 2 == 0 and n1 % 2 == 0

    @functools.partial(
        pl.kernel,
        out_type=jax.ShapeDtypeStruct((NC, NP, D), jnp.float32),
        mesh=_get_mesh(),
        scratch_types=[
            pltpu.VMEM((CHUNK,), jnp.int32),
            pltpu.VMEM((CHUNK,), jnp.int32),
            pltpu.VMEM((CHUNK,), jnp.int32),
            pltpu.VMEM((CHUNK,), jnp.int32),
            pltpu.VMEM((CHUNK, D), jnp.float32),
            pltpu.VMEM((CHUNK, D), jnp.float32),
            pltpu.VMEM_SHARED((NP, D), jnp.float32),
            pltpu.SemaphoreType.DMA,
            pltpu.SemaphoreType.DMA,
            pltpu.SemaphoreType.DMA,
            pltpu.SemaphoreType.DMA,
            pltpu.SemaphoreType.DMA,
            pltpu.SemaphoreType.DMA,
        ],
    )
    def prop_kernel(src_hbm, dst_hbm, g_hbm, zrow_hbm, out_hbm,
                    srcA, srcB, dstA, dstB, rowsA, rowsB, acc_sp,
                    iA, iB, gA, gB, sA, sB):
        c = lax.axis_index("c")
        s = lax.axis_index("s")
        nct = jnp.where(c == 0, n0, n1)
        base_chunk = jnp.where(c == 0, s * n0, NS * n0 + s * n1)
        tile_base = base_chunk * CHUNK
        pltpu.sync_copy(zrow_hbm, acc_sp.at[pl.ds(s * RPT, RPT)])

        def idx_start(sbuf, dbuf, sem, k):
            base = tile_base + k * CHUNK
            pltpu.async_copy(src_hbm.at[pl.ds(base, CHUNK)], sbuf, sem)
            pltpu.async_copy(dst_hbm.at[pl.ds(base, CHUNK)], dbuf, sem)

        def idx_wait(sbuf, dbuf, sem):
            pltpu.make_async_copy(src_hbm.at[pl.ds(0, CHUNK)], sbuf, sem).wait()
            pltpu.make_async_copy(dst_hbm.at[pl.ds(0, CHUNK)], dbuf, sem).wait()

        def g_start(sbuf, rbuf, sem):
            pltpu.async_copy(g_hbm.at[sbuf], rbuf, sem)

        def g_wait(sbuf, rbuf, sem):
            pltpu.make_async_copy(g_hbm.at[sbuf], rbuf, sem).wait()

        def s_start(dbuf, rbuf, sem):
            pltpu.async_copy(rbuf, acc_sp.at[dbuf], sem, add=True)

        def s_wait(dbuf, rbuf, sem):
            pltpu.make_async_copy(rbuf, acc_sp.at[dbuf], sem).wait()

        plsc.subcore_barrier()
        idx_start(srcA, dstA, iA, 0)
        idx_start(srcB, dstB, iB, 1)

        def body(gg, carry):
            k0 = 2 * gg
            idx_wait(srcA, dstA, iA)
            g_start(srcA, rowsA, gA)
            idx_wait(srcB, dstB, iB)
            g_start(srcB, rowsB, gB)
            g_wait(srcA, rowsA, gA)
            s_start(dstA, rowsA, sA)
            g_wait(srcB, rowsB, gB)
            s_start(dstB, rowsB, sB)
            s_wait(dstA, rowsA, sA)

            @pl.when(k0 + 2 < nct)
            def _():
                idx_start(srcA, dstA, iA, k0 + 2)

            s_wait(dstB, rowsB, sB)

            @pl.when(k0 + 3 < nct)
            def _():
                idx_start(srcB, dstB, iB, k0 + 3)

            return carry

        lax.fori_loop(0, nct // 2, body, 0)
        plsc.subcore_barrier()
        pltpu.sync_copy(acc_sp.at[pl.ds(s * RPT, RPT)],
                        out_hbm.at[c, pl.ds(s * RPT, RPT)])

    return prop_kernel


_BR = 2048
_GRID = NP // _BR


def _dinv_from_partials(dp):
    s = jnp.sum(dp, axis=0)
    s = jnp.sum(s, axis=1, keepdims=True)
    return lax.rsqrt(s * (1.0 / 128.0) + 1.0)


def _tc_a_body(dp_ref, x_ref, w_ref, g_ref):
    dinv = _dinv_from_partials(dp_ref[...])
    g_ref[...] = jnp.dot(x_ref[...], w_ref[...],
                         preferred_element_type=jnp.float32) * dinv


def _tc_b_body(dp_ref, p_ref, g_ref, b_ref, w_ref, out_ref):
    dinv = _dinv_from_partials(dp_ref[...])
    p = p_ref[...]
    t = (p[0] + p[1] + g_ref[...]) * dinv + b_ref[...]
    z = jnp.maximum(t, 0.0)
    out_ref[...] = jnp.dot(z, w_ref[...],
                           preferred_element_type=jnp.float32) * dinv


def _tc_c_body(dp_ref, p_ref, g_ref, b_ref, out_ref):
    dinv = _dinv_from_partials(dp_ref[...])
    p = p_ref[...]
    out_ref[...] = (p[0] + p[1] + g_ref[...]) * dinv + b_ref[...]


_dp_spec = pl.BlockSpec((NC, _BR, D), lambda i: (0, i, 0))
_row_spec = pl.BlockSpec((_BR, D), lambda i: (i, 0))
_p_spec = pl.BlockSpec((NC, _BR, D), lambda i: (0, i, 0))
_w_spec = pl.BlockSpec((D, D), lambda i: (0, 0))
_b_spec = pl.BlockSpec((1, D), lambda i: (0, 0))
_out_shape = jax.ShapeDtypeStruct((NP, D), jnp.float32)

_tc_a = pl.pallas_call(
    _tc_a_body, grid=(_GRID,),
    in_specs=[_dp_spec, _row_spec, _w_spec],
    out_specs=_row_spec, out_shape=_out_shape)

_tc_b = pl.pallas_call(
    _tc_b_body, grid=(_GRID,),
    in_specs=[_dp_spec, _p_spec, _row_spec, _b_spec, _w_spec],
    out_specs=_row_spec, out_shape=_out_shape)

_tc_c = pl.pallas_call(
    _tc_c_body, grid=(_GRID,),
    in_specs=[_dp_spec, _p_spec, _row_spec, _b_spec],
    out_specs=_row_spec, out_shape=_out_shape)


_F0 = 0.79


def kernel(x, edge_index, W1, b1, W2, b2):
    e = edge_index.shape[1]
    n0, n1 = _chunk_split(e, _F0)
    ep = NS * (n0 + n1) * CHUNK
    ncht_deg = ep // (NC * NS * CHUNK)

    src = edge_index[0].astype(jnp.int32)
    dst = edge_index[1].astype(jnp.int32)
    pad = jnp.full((ep - e,), N, dtype=jnp.int32)
    srcp = jnp.concatenate([src, pad])
    dstp = jnp.concatenate([dst, pad])
    xp = jnp.pad(x, ((0, NP - x.shape[0]), (0, 0)))

    ones_c = jnp.ones((CHUNK, D), jnp.float32)
    zrow = jnp.zeros((RPT, D), jnp.float32)

    dp = _make_deg_kernel(ncht_deg)(dstp, zrow, ones_c)
    g1 = _tc_a(dp, xp, W1)
    prop = _make_prop_kernel(n0, n1)
    p1 = prop(srcp, dstp, g1, zrow)
    g2 = _tc_b(dp, p1, g1, b1.reshape(1, D), W2)
    p2 = prop(srcp, dstp, g2, zrow)
    out = _tc_c(dp, p2, g2, b2.reshape(1, D))
    return out[:N]


# speedup vs baseline: 1.4027x; 1.3388x over previous
import functools

import jax
import jax.numpy as jnp
from jax import lax
from jax.experimental import pallas as pl
from jax.experimental.pallas import tpu as pltpu
from jax.experimental.pallas import tpu_sc as plsc

N = 10000
D = 128
NP = 10240
NC = 2
NS = 16
CHUNK = 128
RPT = NP // NS

@functools.cache
def _get_mesh():
    return plsc.VectorSubcoreMesh(core_axis_name="c", subcore_axis_name="s",
                                  num_cores=NC, num_subcores=NS)


def _chunk_split(e, f0):
    ct = -(-e // (2 * NS * CHUNK)) * (2 * NS)
    per_tile = ct // NS
    n0 = max(2, min(per_tile - 2, 2 * round(f0 * per_tile / 2.0)))
    return n0, per_tile - n0


def _make_deg_kernel(NCHT):
    @functools.partial(
        pl.kernel,
        out_type=jax.ShapeDtypeStruct((NC, NP, D), jnp.float32),
        mesh=_get_mesh(),
        scratch_types=[
            pltpu.VMEM((CHUNK,), jnp.int32),
            pltpu.VMEM((CHUNK,), jnp.int32),
            pltpu.VMEM((CHUNK, D), jnp.float32),
            pltpu.VMEM_SHARED((NP, D), jnp.float32),
            pltpu.SemaphoreType.DMA,
            pltpu.SemaphoreType.DMA,
            pltpu.SemaphoreType.DMA,
            pltpu.SemaphoreType.DMA,
        ],
    )
    def deg_kernel(dst_hbm, zdeg_hbm, ones_hbm, out_hbm,
                   dstA, dstB, ones_v, deg_sp, iA, iB, sA, sB):
        c = lax.axis_index("c")
        s = lax.axis_index("s")
        w = c * NS + s
        pltpu.sync_copy(zdeg_hbm, deg_sp.at[pl.ds(s * RPT, RPT)])
        pltpu.sync_copy(ones_hbm, ones_v)
        tile_base = w * NCHT * CHUNK

        def idx_start(buf, sem, k):
            pltpu.async_copy(dst_hbm.at[pl.ds(tile_base + k * CHUNK, CHUNK)],
                             buf, sem)

        def idx_wait(buf, sem):
            pltpu.make_async_copy(dst_hbm.at[pl.ds(0, CHUNK)], buf, sem).wait()

        def s_start(buf, sem):
            pltpu.async_copy(ones_v, deg_sp.at[buf], sem, add=True)

        def s_wait(buf, sem):
            pltpu.make_async_copy(ones_v, deg_sp.at[buf], sem).wait()

        plsc.subcore_barrier()
        idx_start(dstA, iA, 0)
        idx_start(dstB, iB, 1)
        NH2 = NCHT // 2

        def body(gg, carry):
            k0 = 2 * gg
            idx_wait(dstA, iA)
            s_start(dstA, sA)
            idx_wait(dstB, iB)
            s_start(dstB, sB)
            s_wait(dstA, sA)

            @pl.when(k0 + 2 < NCHT)
            def _():
                idx_start(dstA, iA, k0 + 2)

            s_wait(dstB, sB)

            @pl.when(k0 + 3 < NCHT)
            def _():
                idx_start(dstB, iB, k0 + 3)

            return carry

        lax.fori_loop(0, NH2, body, 0)
        if NCHT % 2 == 1:
            idx_wait(dstA, iA)
            s_start(dstA, sA)
            s_wait(dstA, sA)
        plsc.subcore_barrier()
        pltpu.sync_copy(deg_sp.at[pl.ds(s * RPT, RPT)],
                        out_hbm.at[c, pl.ds(s * RPT, RPT)])

    return deg_kernel


def _make_prop_kernel(n0, n1):
    assert n0 % 2 == 0 and n1 % 2 == 0

    @functools.partial(
        pl.kernel,
        out_type=jax.ShapeDtypeStruct((NC, NP, D), jnp.float32),
        mesh=_get_mesh(),
        scratch_types=[
            pltpu.VMEM((CHUNK,), jnp.int32),
            pltpu.VMEM((CHUNK,), jnp.int32),
            pltpu.VMEM((CHUNK,), jnp.int32),
            pltpu.VMEM((CHUNK,), jnp.int32),
            pltpu.VMEM((CHUNK, D), jnp.float32),
            pltpu.VMEM((CHUNK, D), jnp.float32),
            pltpu.VMEM_SHARED((NP, D), jnp.float32),
            pltpu.SemaphoreType.DMA,
            pltpu.SemaphoreType.DMA,
            pltpu.SemaphoreType.DMA,
            pltpu.SemaphoreType.DMA,
            pltpu.SemaphoreType.DMA,
            pltpu.SemaphoreType.DMA,
        ],
    )
    def prop_kernel(src_hbm, dst_hbm, g_hbm, zrow_hbm, out_hbm,
                    srcA, srcB, dstA, dstB, rowsA, rowsB, acc_sp,
                    iA, iB, gA, gB, sA, sB):
        c = lax.axis_index("c")
        s = lax.axis_index("s")
        nct = jnp.where(c == 0, n0, n1)
        base_chunk = jnp.where(c == 0, s * n0, NS * n0 + s * n1)
        tile_base = base_chunk * CHUNK
        pltpu.sync_copy(zrow_hbm, acc_sp.at[pl.ds(s * RPT, RPT)])

        def idx_start(sbuf, dbuf, sem, k):
            base = tile_base + k * CHUNK
            pltpu.async_copy(src_hbm.at[pl.ds(base, CHUNK)], sbuf, sem)
            pltpu.async_copy(dst_hbm.at[pl.ds(base, CHUNK)], dbuf, sem)

        def idx_wait(sbuf, dbuf, sem):
            pltpu.make_async_copy(src_hbm.at[pl.ds(0, CHUNK)], sbuf, sem).wait()
            pltpu.make_async_copy(dst_hbm.at[pl.ds(0, CHUNK)], dbuf, sem).wait()

        def g_start(sbuf, rbuf, sem):
            pltpu.async_copy(g_hbm.at[sbuf], rbuf, sem)

        def g_wait(sbuf, rbuf, sem):
            pltpu.make_async_copy(g_hbm.at[sbuf], rbuf, sem).wait()

        def s_start(dbuf, rbuf, sem):
            pltpu.async_copy(rbuf, acc_sp.at[dbuf], sem, add=True)

        def s_wait(dbuf, rbuf, sem):
            pltpu.make_async_copy(rbuf, acc_sp.at[dbuf], sem).wait()

        plsc.subcore_barrier()
        idx_start(srcA, dstA, iA, 0)
        idx_start(srcB, dstB, iB, 1)

        def body(gg, carry):
            k0 = 2 * gg
            idx_wait(srcA, dstA, iA)
            g_start(srcA, rowsA, gA)
            idx_wait(srcB, dstB, iB)
            g_start(srcB, rowsB, gB)
            g_wait(srcA, rowsA, gA)
            s_start(dstA, rowsA, sA)
            g_wait(srcB, rowsB, gB)
            s_start(dstB, rowsB, sB)
            s_wait(dstA, rowsA, sA)

            @pl.when(k0 + 2 < nct)
            def _():
                idx_start(srcA, dstA, iA, k0 + 2)

            s_wait(dstB, rowsB, sB)

            @pl.when(k0 + 3 < nct)
            def _():
                idx_start(srcB, dstB, iB, k0 + 3)

            return carry

        lax.fori_loop(0, nct // 2, body, 0)
        plsc.subcore_barrier()
        pltpu.sync_copy(acc_sp.at[pl.ds(s * RPT, RPT)],
                        out_hbm.at[c, pl.ds(s * RPT, RPT)])

    return prop_kernel


_BR = 2048
_GRID = NP // _BR


def _dinv_from_partials(dp):
    s = jnp.sum(dp, axis=0)
    s = jnp.sum(s, axis=1, keepdims=True)
    return lax.rsqrt(s * (1.0 / 128.0) + 1.0)


def _tc_a_body(dp_ref, x_ref, w_ref, g_ref):
    dinv = _dinv_from_partials(dp_ref[...])
    g_ref[...] = jnp.dot(x_ref[...], w_ref[...],
                         preferred_element_type=jnp.float32) * dinv


def _tc_b_body(dp_ref, p_ref, g_ref, b_ref, w_ref, out_ref):
    dinv = _dinv_from_partials(dp_ref[...])
    p = p_ref[...]
    t = (p[0] + p[1] + g_ref[...]) * dinv + b_ref[...]
    z = jnp.maximum(t, 0.0)
    out_ref[...] = jnp.dot(z, w_ref[...],
                           preferred_element_type=jnp.float32) * dinv


def _tc_c_body(dp_ref, p_ref, g_ref, b_ref, out_ref):
    dinv = _dinv_from_partials(dp_ref[...])
    p = p_ref[...]
    out_ref[...] = (p[0] + p[1] + g_ref[...]) * dinv + b_ref[...]


_dp_spec = pl.BlockSpec((NC, _BR, D), lambda i: (0, i, 0))
_row_spec = pl.BlockSpec((_BR, D), lambda i: (i, 0))
_p_spec = pl.BlockSpec((NC, _BR, D), lambda i: (0, i, 0))
_w_spec = pl.BlockSpec((D, D), lambda i: (0, 0))
_b_spec = pl.BlockSpec((1, D), lambda i: (0, 0))
_out_shape = jax.ShapeDtypeStruct((NP, D), jnp.float32)

_tc_a = pl.pallas_call(
    _tc_a_body, grid=(_GRID,),
    in_specs=[_dp_spec, _row_spec, _w_spec],
    out_specs=_row_spec, out_shape=_out_shape)

_tc_b = pl.pallas_call(
    _tc_b_body, grid=(_GRID,),
    in_specs=[_dp_spec, _p_spec, _row_spec, _b_spec, _w_spec],
    out_specs=_row_spec, out_shape=_out_shape)

_tc_c = pl.pallas_call(
    _tc_c_body, grid=(_GRID,),
    in_specs=[_dp_spec, _p_spec, _row_spec, _b_spec],
    out_specs=_row_spec, out_shape=_out_shape)


_F0 = 0.5


def kernel(x, edge_index, W1, b1, W2, b2):
    e = edge_index.shape[1]
    n0, n1 = _chunk_split(e, _F0)
    ep = NS * (n0 + n1) * CHUNK
    ncht_deg = ep // (NC * NS * CHUNK)

    src = edge_index[0].astype(jnp.int32)
    dst = edge_index[1].astype(jnp.int32)
    pad = N + (jnp.arange(ep - e, dtype=jnp.int32) % (NP - N))
    srcp = jnp.concatenate([src, pad])
    dstp = jnp.concatenate([dst, pad])
    xp = jnp.pad(x, ((0, NP - x.shape[0]), (0, 0)))

    ones_c = jnp.ones((CHUNK, D), jnp.float32)
    zrow = jnp.zeros((RPT, D), jnp.float32)

    dp = _make_deg_kernel(ncht_deg)(dstp, zrow, ones_c)
    g1 = _tc_a(dp, xp, W1)
    prop = _make_prop_kernel(n0, n1)
    p1 = prop(srcp, dstp, g1, zrow)
    g2 = _tc_b(dp, p1, g1, b1.reshape(1, D), W2)
    p2 = prop(srcp, dstp, g2, zrow)
    out = _tc_c(dp, p2, g2, b2.reshape(1, D))
    return out[:N]


# speedup vs baseline: 1.5119x; 1.0779x over previous
import functools

import jax
import jax.numpy as jnp
from jax import lax
from jax.experimental import pallas as pl
from jax.experimental.pallas import tpu as pltpu
from jax.experimental.pallas import tpu_sc as plsc

N = 10000
D = 128
NP = 10240
NC = 2
NS = 16
CHUNK = 96
NSLOT = 3
RPT = NP // NS

@functools.cache
def _get_mesh():
    return plsc.VectorSubcoreMesh(core_axis_name="c", subcore_axis_name="s",
                                  num_cores=NC, num_subcores=NS)


def _chunk_split(e, f0):
    per_tile = 2 * NSLOT * (-(-e // (NS * 2 * NSLOT * CHUNK)))
    n0 = max(NSLOT, min(per_tile - NSLOT,
                        NSLOT * round(f0 * per_tile / NSLOT)))
    return n0, per_tile - n0


def _make_deg_kernel(NCHT):
    @functools.partial(
        pl.kernel,
        out_type=jax.ShapeDtypeStruct((NC, NP, D), jnp.float32),
        mesh=_get_mesh(),
        scratch_types=[
            pltpu.VMEM((CHUNK,), jnp.int32),
            pltpu.VMEM((CHUNK,), jnp.int32),
            pltpu.VMEM((CHUNK, D), jnp.float32),
            pltpu.VMEM_SHARED((NP, D), jnp.float32),
            pltpu.SemaphoreType.DMA,
            pltpu.SemaphoreType.DMA,
            pltpu.SemaphoreType.DMA,
            pltpu.SemaphoreType.DMA,
        ],
    )
    def deg_kernel(dst_hbm, zdeg_hbm, ones_hbm, out_hbm,
                   dstA, dstB, ones_v, deg_sp, iA, iB, sA, sB):
        c = lax.axis_index("c")
        s = lax.axis_index("s")
        w = c * NS + s
        pltpu.sync_copy(zdeg_hbm, deg_sp.at[pl.ds(s * RPT, RPT)])
        pltpu.sync_copy(ones_hbm, ones_v)
        tile_base = w * NCHT * CHUNK

        def idx_start(buf, sem, k):
            pltpu.async_copy(dst_hbm.at[pl.ds(tile_base + k * CHUNK, CHUNK)],
                             buf, sem)

        def idx_wait(buf, sem):
            pltpu.make_async_copy(dst_hbm.at[pl.ds(0, CHUNK)], buf, sem).wait()

        def s_start(buf, sem):
            pltpu.async_copy(ones_v, deg_sp.at[buf], sem, add=True)

        def s_wait(buf, sem):
            pltpu.make_async_copy(ones_v, deg_sp.at[buf], sem).wait()

        plsc.subcore_barrier()
        idx_start(dstA, iA, 0)
        idx_start(dstB, iB, 1)
        NH2 = NCHT // 2

        def body(gg, carry):
            k0 = 2 * gg
            idx_wait(dstA, iA)
            s_start(dstA, sA)
            idx_wait(dstB, iB)
            s_start(dstB, sB)
            s_wait(dstA, sA)

            @pl.when(k0 + 2 < NCHT)
            def _():
                idx_start(dstA, iA, k0 + 2)

            s_wait(dstB, sB)

            @pl.when(k0 + 3 < NCHT)
            def _():
                idx_start(dstB, iB, k0 + 3)

            return carry

        lax.fori_loop(0, NH2, body, 0)
        if NCHT % 2 == 1:
            idx_wait(dstA, iA)
            s_start(dstA, sA)
            s_wait(dstA, sA)
        plsc.subcore_barrier()
        pltpu.sync_copy(deg_sp.at[pl.ds(s * RPT, RPT)],
                        out_hbm.at[c, pl.ds(s * RPT, RPT)])

    return deg_kernel


def _make_prop_kernel(n0, n1):
    assert n0 % NSLOT == 0 and n1 % NSLOT == 0

    @functools.partial(
        pl.kernel,
        out_type=jax.ShapeDtypeStruct((NC, NP, D), jnp.float32),
        mesh=_get_mesh(),
        scratch_types=(
            [pltpu.VMEM((CHUNK,), jnp.int32)] * NSLOT
            + [pltpu.VMEM((CHUNK,), jnp.int32)] * NSLOT
            + [pltpu.VMEM((CHUNK, D), jnp.float32)] * NSLOT
            + [pltpu.VMEM_SHARED((NP, D), jnp.float32)]
            + [pltpu.SemaphoreType.DMA] * (3 * NSLOT)
        ),
    )
    def prop_kernel(src_hbm, dst_hbm, g_hbm, zrow_hbm, out_hbm, *sc):
        srcs = sc[0:NSLOT]
        dsts = sc[NSLOT:2 * NSLOT]
        rows = sc[2 * NSLOT:3 * NSLOT]
        acc_sp = sc[3 * NSLOT]
        isem = sc[3 * NSLOT + 1:4 * NSLOT + 1]
        gsem = sc[4 * NSLOT + 1:5 * NSLOT + 1]
        ssem = sc[5 * NSLOT + 1:6 * NSLOT + 1]
        c = lax.axis_index("c")
        s = lax.axis_index("s")
        nct = jnp.where(c == 0, n0, n1)
        base_chunk = jnp.where(c == 0, s * n0, NS * n0 + s * n1)
        tile_base = base_chunk * CHUNK
        pltpu.sync_copy(zrow_hbm, acc_sp.at[pl.ds(s * RPT, RPT)])

        def idx_start(j, k):
            base = tile_base + k * CHUNK
            pltpu.async_copy(src_hbm.at[pl.ds(base, CHUNK)], srcs[j], isem[j])
            pltpu.async_copy(dst_hbm.at[pl.ds(base, CHUNK)], dsts[j], isem[j])

        def idx_wait(j):
            pltpu.make_async_copy(src_hbm.at[pl.ds(0, CHUNK)], srcs[j], isem[j]).wait()
            pltpu.make_async_copy(dst_hbm.at[pl.ds(0, CHUNK)], dsts[j], isem[j]).wait()

        def g_start(j):
            pltpu.async_copy(g_hbm.at[srcs[j]], rows[j], gsem[j])

        def g_wait(j):
            pltpu.make_async_copy(g_hbm.at[srcs[j]], rows[j], gsem[j]).wait()

        def s_start(j):
            pltpu.async_copy(rows[j], acc_sp.at[dsts[j]], ssem[j], add=True)

        def s_wait(j):
            pltpu.make_async_copy(rows[j], acc_sp.at[dsts[j]], ssem[j]).wait()

        plsc.subcore_barrier()
        for j in range(NSLOT):
            idx_start(j, j)

        def body(gg, carry):
            k0 = NSLOT * gg
            for j in range(NSLOT):
                idx_wait(j)
                g_start(j)
            for j in range(NSLOT):
                g_wait(j)
                s_start(j)
            for j in range(NSLOT):
                s_wait(j)

                @pl.when(k0 + NSLOT + j < nct)
                def _():
                    idx_start(j, k0 + NSLOT + j)

            return carry

        lax.fori_loop(0, nct // NSLOT, body, 0)
        plsc.subcore_barrier()
        pltpu.sync_copy(acc_sp.at[pl.ds(s * RPT, RPT)],
                        out_hbm.at[c, pl.ds(s * RPT, RPT)])

    return prop_kernel


_BR = 2048
_GRID = NP // _BR


def _dinv_from_partials(dp):
    s = jnp.sum(dp, axis=0)
    s = jnp.sum(s, axis=1, keepdims=True)
    return lax.rsqrt(s * (1.0 / 128.0) + 1.0)


def _tc_a_body(dp_ref, x_ref, w_ref, g_ref):
    dinv = _dinv_from_partials(dp_ref[...])
    g_ref[...] = jnp.dot(x_ref[...], w_ref[...],
                         preferred_element_type=jnp.float32) * dinv


def _tc_b_body(dp_ref, p_ref, g_ref, b_ref, w_ref, out_ref):
    dinv = _dinv_from_partials(dp_ref[...])
    p = p_ref[...]
    t = (p[0] + p[1] + g_ref[...]) * dinv + b_ref[...]
    z = jnp.maximum(t, 0.0)
    out_ref[...] = jnp.dot(z, w_ref[...],
                           preferred_element_type=jnp.float32) * dinv


def _tc_c_body(dp_ref, p_ref, g_ref, b_ref, out_ref):
    dinv = _dinv_from_partials(dp_ref[...])
    p = p_ref[...]
    out_ref[...] = (p[0] + p[1] + g_ref[...]) * dinv + b_ref[...]


_dp_spec = pl.BlockSpec((NC, _BR, D), lambda i: (0, i, 0))
_row_spec = pl.BlockSpec((_BR, D), lambda i: (i, 0))
_p_spec = pl.BlockSpec((NC, _BR, D), lambda i: (0, i, 0))
_w_spec = pl.BlockSpec((D, D), lambda i: (0, 0))
_b_spec = pl.BlockSpec((1, D), lambda i: (0, 0))
_out_shape = jax.ShapeDtypeStruct((NP, D), jnp.float32)

_tc_a = pl.pallas_call(
    _tc_a_body, grid=(_GRID,),
    in_specs=[_dp_spec, _row_spec, _w_spec],
    out_specs=_row_spec, out_shape=_out_shape)

_tc_b = pl.pallas_call(
    _tc_b_body, grid=(_GRID,),
    in_specs=[_dp_spec, _p_spec, _row_spec, _b_spec, _w_spec],
    out_specs=_row_spec, out_shape=_out_shape)

_tc_c = pl.pallas_call(
    _tc_c_body, grid=(_GRID,),
    in_specs=[_dp_spec, _p_spec, _row_spec, _b_spec],
    out_specs=_row_spec, out_shape=_out_shape)


_F0 = 0.5


def kernel(x, edge_index, W1, b1, W2, b2):
    e = edge_index.shape[1]
    n0, n1 = _chunk_split(e, _F0)
    ep = NS * (n0 + n1) * CHUNK
    ncht_deg = ep // (NC * NS * CHUNK)

    src = edge_index[0].astype(jnp.int32)
    dst = edge_index[1].astype(jnp.int32)
    pad = N + (jnp.arange(ep - e, dtype=jnp.int32) % (NP - N))
    srcp = jnp.concatenate([src, pad])
    dstp = jnp.concatenate([dst, pad])
    xp = jnp.pad(x, ((0, NP - x.shape[0]), (0, 0)))

    ones_c = jnp.ones((CHUNK, D), jnp.float32)
    zrow = jnp.zeros((RPT, D), jnp.float32)

    dp = _make_deg_kernel(ncht_deg)(dstp, zrow, ones_c)
    g1 = _tc_a(dp, xp, W1)
    prop = _make_prop_kernel(n0, n1)
    p1 = prop(srcp, dstp, g1, zrow)
    g2 = _tc_b(dp, p1, g1, b1.reshape(1, D), W2)
    p2 = prop(srcp, dstp, g2, zrow)
    out = _tc_c(dp, p2, g2, b2.reshape(1, D))
    return out[:N]


# speedup vs baseline: 1.5308x; 1.0125x over previous
import functools

import jax
import jax.numpy as jnp
from jax import lax
from jax.experimental import pallas as pl
from jax.experimental.pallas import tpu as pltpu
from jax.experimental.pallas import tpu_sc as plsc

N = 10000
D = 128
NP = 10240
NC = 2
NS = 16
CHUNK = 112
NSLOT = 3
RPT = NP // NS

@functools.cache
def _get_mesh():
    return plsc.VectorSubcoreMesh(core_axis_name="c", subcore_axis_name="s",
                                  num_cores=NC, num_subcores=NS)


def _chunk_split(e, f0):
    per_tile = 2 * NSLOT * (-(-e // (NS * 2 * NSLOT * CHUNK)))
    n0 = max(NSLOT, min(per_tile - NSLOT,
                        NSLOT * round(f0 * per_tile / NSLOT)))
    return n0, per_tile - n0


def _make_deg_kernel(NCHT):
    @functools.partial(
        pl.kernel,
        out_type=jax.ShapeDtypeStruct((NC, NP, D), jnp.float32),
        mesh=_get_mesh(),
        scratch_types=[
            pltpu.VMEM((CHUNK,), jnp.int32),
            pltpu.VMEM((CHUNK,), jnp.int32),
            pltpu.VMEM((CHUNK, D), jnp.float32),
            pltpu.VMEM_SHARED((NP, D), jnp.float32),
            pltpu.SemaphoreType.DMA,
            pltpu.SemaphoreType.DMA,
            pltpu.SemaphoreType.DMA,
            pltpu.SemaphoreType.DMA,
        ],
    )
    def deg_kernel(dst_hbm, zdeg_hbm, ones_hbm, out_hbm,
                   dstA, dstB, ones_v, deg_sp, iA, iB, sA, sB):
        c = lax.axis_index("c")
        s = lax.axis_index("s")
        w = c * NS + s
        pltpu.sync_copy(zdeg_hbm, deg_sp.at[pl.ds(s * RPT, RPT)])
        pltpu.sync_copy(ones_hbm, ones_v)
        tile_base = w * NCHT * CHUNK

        def idx_start(buf, sem, k):
            pltpu.async_copy(dst_hbm.at[pl.ds(tile_base + k * CHUNK, CHUNK)],
                             buf, sem)

        def idx_wait(buf, sem):
            pltpu.make_async_copy(dst_hbm.at[pl.ds(0, CHUNK)], buf, sem).wait()

        def s_start(buf, sem):
            pltpu.async_copy(ones_v, deg_sp.at[buf], sem, add=True)

        def s_wait(buf, sem):
            pltpu.make_async_copy(ones_v, deg_sp.at[buf], sem).wait()

        plsc.subcore_barrier()
        idx_start(dstA, iA, 0)
        idx_start(dstB, iB, 1)
        NH2 = NCHT // 2

        def body(gg, carry):
            k0 = 2 * gg
            idx_wait(dstA, iA)
            s_start(dstA, sA)
            idx_wait(dstB, iB)
            s_start(dstB, sB)
            s_wait(dstA, sA)

            @pl.when(k0 + 2 < NCHT)
            def _():
                idx_start(dstA, iA, k0 + 2)

            s_wait(dstB, sB)

            @pl.when(k0 + 3 < NCHT)
            def _():
                idx_start(dstB, iB, k0 + 3)

            return carry

        lax.fori_loop(0, NH2, body, 0)
        if NCHT % 2 == 1:
            idx_wait(dstA, iA)
            s_start(dstA, sA)
            s_wait(dstA, sA)
        plsc.subcore_barrier()
        pltpu.sync_copy(deg_sp.at[pl.ds(s * RPT, RPT)],
                        out_hbm.at[c, pl.ds(s * RPT, RPT)])

    return deg_kernel


def _make_prop_kernel(n0, n1):
    assert n0 % NSLOT == 0 and n1 % NSLOT == 0

    @functools.partial(
        pl.kernel,
        out_type=jax.ShapeDtypeStruct((NC, NP, D), jnp.float32),
        mesh=_get_mesh(),
        scratch_types=(
            [pltpu.VMEM((CHUNK,), jnp.int32)] * NSLOT
            + [pltpu.VMEM((CHUNK,), jnp.int32)] * NSLOT
            + [pltpu.VMEM((CHUNK, D), jnp.float32)] * NSLOT
            + [pltpu.VMEM_SHARED((NP, D), jnp.float32)]
            + [pltpu.SemaphoreType.DMA] * (3 * NSLOT)
        ),
    )
    def prop_kernel(src_hbm, dst_hbm, g_hbm, zrow_hbm, out_hbm, *sc):
        srcs = sc[0:NSLOT]
        dsts = sc[NSLOT:2 * NSLOT]
        rows = sc[2 * NSLOT:3 * NSLOT]
        acc_sp = sc[3 * NSLOT]
        isem = sc[3 * NSLOT + 1:4 * NSLOT + 1]
        gsem = sc[4 * NSLOT + 1:5 * NSLOT + 1]
        ssem = sc[5 * NSLOT + 1:6 * NSLOT + 1]
        c = lax.axis_index("c")
        s = lax.axis_index("s")
        nct = jnp.where(c == 0, n0, n1)
        base_chunk = jnp.where(c == 0, s * n0, NS * n0 + s * n1)
        tile_base = base_chunk * CHUNK
        pltpu.sync_copy(zrow_hbm, acc_sp.at[pl.ds(s * RPT, RPT)])

        def idx_start(j, k):
            base = tile_base + k * CHUNK
            pltpu.async_copy(src_hbm.at[pl.ds(base, CHUNK)], srcs[j], isem[j])
            pltpu.async_copy(dst_hbm.at[pl.ds(base, CHUNK)], dsts[j], isem[j])

        def idx_wait(j):
            pltpu.make_async_copy(src_hbm.at[pl.ds(0, CHUNK)], srcs[j], isem[j]).wait()
            pltpu.make_async_copy(dst_hbm.at[pl.ds(0, CHUNK)], dsts[j], isem[j]).wait()

        def g_start(j):
            pltpu.async_copy(g_hbm.at[srcs[j]], rows[j], gsem[j])

        def g_wait(j):
            pltpu.make_async_copy(g_hbm.at[srcs[j]], rows[j], gsem[j]).wait()

        def s_start(j):
            pltpu.async_copy(rows[j], acc_sp.at[dsts[j]], ssem[j], add=True)

        def s_wait(j):
            pltpu.make_async_copy(rows[j], acc_sp.at[dsts[j]], ssem[j]).wait()

        plsc.subcore_barrier()
        for j in range(NSLOT):
            idx_start(j, j)

        def body(gg, carry):
            k0 = NSLOT * gg
            for j in range(NSLOT):
                idx_wait(j)
                g_start(j)
            for j in range(NSLOT):
                g_wait(j)
                s_start(j)
            for j in range(NSLOT):
                s_wait(j)

                @pl.when(k0 + NSLOT + j < nct)
                def _():
                    idx_start(j, k0 + NSLOT + j)

            return carry

        lax.fori_loop(0, nct // NSLOT, body, 0)
        plsc.subcore_barrier()
        pltpu.sync_copy(acc_sp.at[pl.ds(s * RPT, RPT)],
                        out_hbm.at[c, pl.ds(s * RPT, RPT)])

    return prop_kernel


_BR = 2048
_GRID = NP // _BR


def _dinv_from_partials(dp):
    s = jnp.sum(dp, axis=0)
    s = jnp.sum(s, axis=1, keepdims=True)
    return lax.rsqrt(s * (1.0 / 128.0) + 1.0)


def _tc_a_body(dp_ref, x_ref, w_ref, g_ref):
    dinv = _dinv_from_partials(dp_ref[...])
    g_ref[...] = jnp.dot(x_ref[...], w_ref[...],
                         preferred_element_type=jnp.float32) * dinv


def _tc_b_body(dp_ref, p_ref, g_ref, b_ref, w_ref, out_ref):
    dinv = _dinv_from_partials(dp_ref[...])
    p = p_ref[...]
    t = (p[0] + p[1] + g_ref[...]) * dinv + b_ref[...]
    z = jnp.maximum(t, 0.0)
    out_ref[...] = jnp.dot(z, w_ref[...],
                           preferred_element_type=jnp.float32) * dinv


def _tc_c_body(dp_ref, p_ref, g_ref, b_ref, out_ref):
    dinv = _dinv_from_partials(dp_ref[...])
    p = p_ref[...]
    out_ref[...] = (p[0] + p[1] + g_ref[...]) * dinv + b_ref[...]


_dp_spec = pl.BlockSpec((NC, _BR, D), lambda i: (0, i, 0))
_row_spec = pl.BlockSpec((_BR, D), lambda i: (i, 0))
_p_spec = pl.BlockSpec((NC, _BR, D), lambda i: (0, i, 0))
_w_spec = pl.BlockSpec((D, D), lambda i: (0, 0))
_b_spec = pl.BlockSpec((1, D), lambda i: (0, 0))
_out_shape = jax.ShapeDtypeStruct((NP, D), jnp.float32)

_tc_a = pl.pallas_call(
    _tc_a_body, grid=(_GRID,),
    in_specs=[_dp_spec, _row_spec, _w_spec],
    out_specs=_row_spec, out_shape=_out_shape)

_tc_b = pl.pallas_call(
    _tc_b_body, grid=(_GRID,),
    in_specs=[_dp_spec, _p_spec, _row_spec, _b_spec, _w_spec],
    out_specs=_row_spec, out_shape=_out_shape)

_tc_c = pl.pallas_call(
    _tc_c_body, grid=(_GRID,),
    in_specs=[_dp_spec, _p_spec, _row_spec, _b_spec],
    out_specs=_row_spec, out_shape=_out_shape)


_F0 = 0.5


def kernel(x, edge_index, W1, b1, W2, b2):
    e = edge_index.shape[1]
    n0, n1 = _chunk_split(e, _F0)
    ep = NS * (n0 + n1) * CHUNK
    ncht_deg = ep // (NC * NS * CHUNK)

    src = edge_index[0].astype(jnp.int32)
    dst = edge_index[1].astype(jnp.int32)
    pad = N + (jnp.arange(ep - e, dtype=jnp.int32) % (NP - N))
    srcp = jnp.concatenate([src, pad])
    dstp = jnp.concatenate([dst, pad])
    xp = jnp.pad(x, ((0, NP - x.shape[0]), (0, 0)))

    ones_c = jnp.ones((CHUNK, D), jnp.float32)
    zrow = jnp.zeros((RPT, D), jnp.float32)

    dp = _make_deg_kernel(ncht_deg)(dstp, zrow, ones_c)
    g1 = _tc_a(dp, xp, W1)
    prop = _make_prop_kernel(n0, n1)
    p1 = prop(srcp, dstp, g1, zrow)
    g2 = _tc_b(dp, p1, g1, b1.reshape(1, D), W2)
    p2 = prop(srcp, dstp, g2, zrow)
    out = _tc_c(dp, p2, g2, b2.reshape(1, D))
    return out[:N]
